# baseline (speedup 1.0000x reference)
import functools

import jax
import jax.numpy as jnp
from jax import lax
from jax.experimental import pallas as pl
from jax.experimental.pallas import tpu as pltpu
from jax.experimental.pallas import tpu_sc as plsc

N = 10000
E = 320000
T = 16
F_IN = 128
H = 128
G1 = 256
G2 = 128

NC = 2
NS = 16
NW = NC * NS
N_PAD = 10240
E_PAD = 327680
E_PER_TILE = E_PAD // NW
EB = 128
NBATCH = E_PER_TILE // EB
ROWS_PER_TILE = N_PAD // NS

R = 1000
NB = N // R

_INTERPRET = False


def _dot_t(a, w):
    return lax.dot_general(a, w, (((1,), (1,)), ((), ())),
                           preferred_element_type=jnp.float32)


def _lstm_body(x_ref, n_ref, wih_ref, whh_ref, b_ref, out_ref):
    x = x_ref[:]
    wih = wih_ref[:]
    whh = whh_ref[:]
    xw = _dot_t(x, wih) + b_ref[:]
    h = jnp.zeros((R, H), jnp.float32)
    c = jnp.zeros((R, H), jnp.float32)
    for t in range(T):
        nt = n_ref[:, t * F_IN:(t + 1) * F_IN]
        g = xw + _dot_t(nt, wih) + _dot_t(h, whh)
        i = jax.nn.sigmoid(g[:, :H])
        f = jax.nn.sigmoid(g[:, H:2 * H])
        gg = jnp.tanh(g[:, 2 * H:3 * H])
        o = jax.nn.sigmoid(g[:, 3 * H:])
        c = f * c + i * gg
        h = o * jnp.tanh(c)
    out_ref[:] = h


def _tc_lstm(x, noise2d, W_ih, W_hh, b_lstm):
    return pl.pallas_call(
        _lstm_body,
        grid=(NB,),
        in_specs=[
            pl.BlockSpec((R, F_IN), lambda i: (i, 0)),
            pl.BlockSpec((R, T * F_IN), lambda i: (i, 0)),
            pl.BlockSpec((4 * H, F_IN), lambda i: (0, 0)),
            pl.BlockSpec((4 * H, H), lambda i: (0, 0)),
            pl.BlockSpec((1, 4 * H), lambda i: (0, 0)),
        ],
        out_specs=pl.BlockSpec((R, H), lambda i: (i, 0)),
        out_shape=jax.ShapeDtypeStruct((N, H), jnp.float32),
        interpret=_INTERPRET,
    )(x, noise2d, W_ih, W_hh, b_lstm)


def _scale1_body(h_ref, w1_ref, dg0_ref, dg1_ref, tab_ref, dinv_ref):
    deg = dg0_ref[:] + dg1_ref[:] + 1.0
    dinv = lax.rsqrt(deg)
    y = jnp.dot(h_ref[:], w1_ref[:], preferred_element_type=jnp.float32)
    ys = y * dinv
    tab_ref[0] = ys[:, :H]
    tab_ref[1] = ys[:, H:]
    dinv_ref[:] = dinv


def _tc_scale1(h_last, W1, dg0, dg1):
    return pl.pallas_call(
        _scale1_body,
        grid=(NB,),
        in_specs=[
            pl.BlockSpec((R, H), lambda i: (i, 0)),
            pl.BlockSpec((H, G1), lambda i: (0, 0)),
            pl.BlockSpec((R, 1), lambda i: (i, 0)),
            pl.BlockSpec((R, 1), lambda i: (i, 0)),
        ],
        out_specs=[
            pl.BlockSpec((2, R, H), lambda i: (0, i, 0)),
            pl.BlockSpec((R, 1), lambda i: (i, 0)),
        ],
        out_shape=[
            jax.ShapeDtypeStruct((2, N, H), jnp.float32),
            jax.ShapeDtypeStruct((N, 1), jnp.float32),
        ],
        interpret=_INTERPRET,
    )(h_last, W1, dg0, dg1)


def _post1_body(a0_ref, a1_ref, c0_ref, c1_ref, ysa_ref, ysb_ref,
                dinv_ref, w2_ref, b1_ref, ys2_ref):
    dinv = dinv_ref[:]
    b1 = b1_ref[:]
    z1a = jax.nn.relu((a0_ref[:] + a1_ref[:] + ysa_ref[:]) * dinv
                      + b1[:, :H])
    z1b = jax.nn.relu((c0_ref[:] + c1_ref[:] + ysb_ref[:]) * dinv
                      + b1[:, H:])
    z1 = jnp.concatenate([z1a, z1b], axis=1)
    ys2 = jnp.dot(z1, w2_ref[:], preferred_element_type=jnp.float32) * dinv
    ys2_ref[:] = ys2


def _tc_post1(a0, a1, c0, c1, ysa, ysb, dinv, W2, b1):
    rblk = pl.BlockSpec((R, H), lambda i: (i, 0))
    return pl.pallas_call(
        _post1_body,
        grid=(NB,),
        in_specs=[rblk, rblk, rblk, rblk, rblk, rblk,
                  pl.BlockSpec((R, 1), lambda i: (i, 0)),
                  pl.BlockSpec((G1, G2), lambda i: (0, 0)),
                  pl.BlockSpec((1, G1), lambda i: (0, 0))],
        out_specs=pl.BlockSpec((R, G2), lambda i: (i, 0)),
        out_shape=jax.ShapeDtypeStruct((N, G2), jnp.float32),
        interpret=_INTERPRET,
    )(a0, a1, c0, c1, ysa, ysb, dinv, W2, b1)


def _final_body(a0_ref, a1_ref, ys2_ref, dinv_ref, b2_ref, wout_ref,
                bout_ref, out_ref, acc_ref):
    i = pl.program_id(0)
    z2 = jax.nn.relu((a0_ref[:] + a1_ref[:] + ys2_ref[:]) * dinv_ref[:]
                     + b2_ref[:])
    p = jnp.sum(z2, axis=0, keepdims=True)

    @pl.when(i == 0)
    def _():
        acc_ref[:] = p

    @pl.when(i != 0)
    def _():
        acc_ref[:] = acc_ref[:] + p

    @pl.when(i == NB - 1)
    def _():
        pooled = acc_ref[:] * (1.0 / N)
        out_ref[:] = jnp.dot(pooled, wout_ref[:],
                             preferred_element_type=jnp.float32) + bout_ref[:]


def _tc_final(a0, a1, ys2, dinv, b2, W_out, b_out2d):
    rblk = pl.BlockSpec((R, G2), lambda i: (i, 0))
    return pl.pallas_call(
        _final_body,
        grid=(NB,),
        in_specs=[rblk, rblk, rblk,
                  pl.BlockSpec((R, 1), lambda i: (i, 0)),
                  pl.BlockSpec((1, G2), lambda i: (0, 0)),
                  pl.BlockSpec((G2, 1), lambda i: (0, 0)),
                  pl.BlockSpec((1, 1), lambda i: (0, 0))],
        out_specs=pl.BlockSpec((1, 1), lambda i: (0, 0)),
        out_shape=jax.ShapeDtypeStruct((1, 1), jnp.float32),
        scratch_shapes=[pltpu.VMEM((1, G2), jnp.float32)],
        interpret=_INTERPRET,
    )(a0, a1, ys2, dinv, b2, W_out, b_out2d)


_SC_MESH = dict(core_axis_name="c", subcore_axis_name="s",
                num_cores=NC, num_subcores=NS)


def _sc_deg(dst2d, zeros128, ones128):
    mesh = plsc.VectorSubcoreMesh(**_SC_MESH)

    @functools.partial(
        pl.kernel,
        out_type=jax.ShapeDtypeStruct((NC, N_PAD, H), jnp.float32),
        mesh=mesh,
        scratch_types=[
            pltpu.VMEM((CB, EB), jnp.int32),
            pltpu.VMEM((EB, H), jnp.float32),
            pltpu.VMEM_SHARED((N_PAD, H), jnp.float32),
        ],
    )
    def k(dst_hbm, zeros_hbm, ones_hbm, out_hbm, idx_d, ones_v, acc):
        cid = lax.axis_index("c")
        sid = lax.axis_index("s")
        wid = cid * NS + sid
        pltpu.sync_copy(zeros_hbm.at[pl.ds(sid * ROWS_PER_TILE, ROWS_PER_TILE)],
                        acc.at[pl.ds(sid * ROWS_PER_TILE, ROWS_PER_TILE)])
        pltpu.sync_copy(ones_hbm, ones_v)
        plsc.subcore_barrier()
        bbase = wid * NBATCH

        def chunk(ch, carry):
            pltpu.sync_copy(dst_hbm.at[pl.ds(bbase + ch * CB, CB)], idx_d)

            def body(j, c2):
                pltpu.sync_copy(ones_v, acc.at[idx_d.at[j]], add=True)
                return c2

            lax.fori_loop(0, CB, body, carry)
            return carry

        lax.fori_loop(0, NBATCH // CB, chunk, 0)
        plsc.subcore_barrier()
        pltpu.sync_copy(
            acc.at[pl.ds(sid * ROWS_PER_TILE, ROWS_PER_TILE)],
            out_hbm.at[cid].at[pl.ds(sid * ROWS_PER_TILE, ROWS_PER_TILE)])

    return k(dst2d, zeros128, ones128)


CB = 40


def _agg_pipeline(table_view, src2d_hbm, dst2d_hbm, acc, idx_s, idx_d,
                  buf0, buf1, sem0, sem1, bbase, nbatch):
    def chunk(ch, carry):
        pltpu.sync_copy(src2d_hbm.at[pl.ds(bbase + ch * CB, CB)], idx_s)
        pltpu.sync_copy(dst2d_hbm.at[pl.ds(bbase + ch * CB, CB)], idx_d)
        pltpu.async_copy(table_view.at[idx_s.at[0]], buf0, sem0)

        def body(jj, c2):
            b0 = jj * 2
            pltpu.make_async_copy(table_view.at[idx_s.at[b0]],
                                  buf0, sem0).wait()
            pltpu.async_copy(table_view.at[idx_s.at[b0 + 1]], buf1, sem1)
            pltpu.sync_copy(buf0, acc.at[idx_d.at[b0]], add=True)
            pltpu.make_async_copy(table_view.at[idx_s.at[b0 + 1]],
                                  buf1, sem1).wait()

            @pl.when(b0 + 2 < CB)
            def _():
                pltpu.async_copy(table_view.at[idx_s.at[b0 + 2]], buf0, sem0)

            pltpu.sync_copy(buf1, acc.at[idx_d.at[b0 + 1]], add=True)
            return c2

        lax.fori_loop(0, CB // 2, body, carry)
        return carry

    lax.fori_loop(0, nbatch // CB, chunk, 0)


_AGG_SCRATCH = [
    pltpu.VMEM((CB, EB), jnp.int32),
    pltpu.VMEM((CB, EB), jnp.int32),
    pltpu.VMEM((EB, H), jnp.float32),
    pltpu.VMEM((EB, H), jnp.float32),
    pltpu.SemaphoreType.DMA,
    pltpu.SemaphoreType.DMA,
    pltpu.VMEM_SHARED((N_PAD, H), jnp.float32),
]


def _sc_agg_pair(tab2, src2d, dst2d, zeros128):
    nbatch = E_PAD // EB // NS
    mesh = plsc.VectorSubcoreMesh(**_SC_MESH)

    @functools.partial(
        pl.kernel,
        out_type=jax.ShapeDtypeStruct((NC, N_PAD, H), jnp.float32),
        mesh=mesh,
        scratch_types=_AGG_SCRATCH,
    )
    def k(tab_hbm, src_hbm, dst_hbm, zeros_hbm, out_hbm,
          idx_s, idx_d, buf0, buf1, sem0, sem1, acc):
        cid = lax.axis_index("c")
        sid = lax.axis_index("s")
        pltpu.sync_copy(zeros_hbm.at[pl.ds(sid * ROWS_PER_TILE, ROWS_PER_TILE)],
                        acc.at[pl.ds(sid * ROWS_PER_TILE, ROWS_PER_TILE)])
        plsc.subcore_barrier()
        _agg_pipeline(tab_hbm.at[cid], src_hbm, dst_hbm, acc, idx_s, idx_d,
                      buf0, buf1, sem0, sem1, sid * nbatch, nbatch)
        plsc.subcore_barrier()
        pltpu.sync_copy(
            acc.at[pl.ds(sid * ROWS_PER_TILE, ROWS_PER_TILE)],
            out_hbm.at[cid].at[pl.ds(sid * ROWS_PER_TILE, ROWS_PER_TILE)])

    return k(tab2, src2d, dst2d, zeros128)


def _sc_agg(table, src2d, dst2d, zeros128):
    nbatch = NBATCH
    mesh = plsc.VectorSubcoreMesh(**_SC_MESH)

    @functools.partial(
        pl.kernel,
        out_type=jax.ShapeDtypeStruct((NC, N_PAD, H), jnp.float32),
        mesh=mesh,
        scratch_types=_AGG_SCRATCH,
    )
    def k(tab_hbm, src_hbm, dst_hbm, zeros_hbm, out_hbm,
          idx_s, idx_d, buf0, buf1, sem0, sem1, acc):
        cid = lax.axis_index("c")
        sid = lax.axis_index("s")
        wid = cid * NS + sid
        pltpu.sync_copy(zeros_hbm.at[pl.ds(sid * ROWS_PER_TILE, ROWS_PER_TILE)],
                        acc.at[pl.ds(sid * ROWS_PER_TILE, ROWS_PER_TILE)])
        plsc.subcore_barrier()
        _agg_pipeline(tab_hbm, src_hbm, dst_hbm, acc, idx_s, idx_d,
                      buf0, buf1, sem0, sem1, wid * nbatch, nbatch)
        plsc.subcore_barrier()
        pltpu.sync_copy(
            acc.at[pl.ds(sid * ROWS_PER_TILE, ROWS_PER_TILE)],
            out_hbm.at[cid].at[pl.ds(sid * ROWS_PER_TILE, ROWS_PER_TILE)])

    return k(table, src2d, dst2d, zeros128)


def kernel(x, W_ih, W_hh, b_ih, b_hh, W1, b1, W2, b2, W_out, b_out,
           edge_index):
    noise2d = (jax.random.normal(jax.random.key(42), (N, T, F_IN),
                                 dtype=jnp.float32) * 0.1).reshape(N, T * F_IN)
    b_lstm = (b_ih + b_hh).reshape(1, 4 * H)

    src = edge_index[0]
    dst = edge_index[1]
    src_p = jnp.concatenate(
        [src, jnp.zeros((E_PAD - E,), jnp.int32)])
    dst_p = jnp.concatenate(
        [dst, jnp.full((E_PAD - E,), N_PAD - 1, jnp.int32)])

    src2d = src_p.reshape(E_PAD // EB, EB)
    dst2d = dst_p.reshape(E_PAD // EB, EB)

    zeros128 = jnp.zeros((N_PAD, H), jnp.float32)
    ones128 = jnp.ones((EB, H), jnp.float32)

    degp = _sc_deg(dst2d, zeros128, ones128)
    dg0 = degp[0, :N, 0:1]
    dg1 = degp[1, :N, 0:1]

    h_last = _tc_lstm(x, noise2d, W_ih, W_hh, b_lstm)

    tab2, dinv = _tc_scale1(h_last, W1, dg0, dg1)

    agg_a = _sc_agg(tab2[0], src2d, dst2d, zeros128)
    agg_b = _sc_agg(tab2[1], src2d, dst2d, zeros128)

    ys2 = _tc_post1(agg_a[0, :N], agg_a[1, :N], agg_b[0, :N], agg_b[1, :N],
                    tab2[0, :N], tab2[1, :N], dinv, W2, b1.reshape(1, G1))

    agg2 = _sc_agg(ys2, src2d, dst2d, zeros128)

    out = _tc_final(agg2[0, :N], agg2[1, :N], ys2, dinv,
                    b2.reshape(1, G2), W_out, b_out.reshape(1, 1))
    return out.reshape(1)


# speedup vs baseline: 1.2409x; 1.2409x over previous
import functools

import jax
import jax.numpy as jnp
from jax import lax
from jax.experimental import pallas as pl
from jax.experimental.pallas import tpu as pltpu
from jax.experimental.pallas import tpu_sc as plsc

N = 10000
E = 320000
T = 16
F_IN = 128
H = 128
G1 = 256
G2 = 128

NC = 2
NS = 16
NW = NC * NS
N_PAD = 10240
E_PAD = 327680
E_PER_TILE = E_PAD // NW
EB = 128
NBATCH = E_PER_TILE // EB
ROWS_PER_TILE = N_PAD // NS

R = 1000
NB = N // R

_INTERPRET = False


def _dot_t(a, w):
    return lax.dot_general(a, w, (((1,), (1,)), ((), ())),
                           preferred_element_type=jnp.float32)


def _lstm_body(x_ref, n_ref, wih_ref, whh_ref, b_ref, out_ref):
    x = x_ref[:]
    wih = wih_ref[:]
    whh = whh_ref[:]
    xw = _dot_t(x, wih) + b_ref[:]
    h = jnp.zeros((R, H), jnp.float32)
    c = jnp.zeros((R, H), jnp.float32)
    for t in range(T):
        nt = n_ref[:, t * F_IN:(t + 1) * F_IN]
        g = xw + _dot_t(nt, wih) + _dot_t(h.astype(jnp.bfloat16), whh)
        i = jax.nn.sigmoid(g[:, :H])
        f = jax.nn.sigmoid(g[:, H:2 * H])
        gg = jnp.tanh(g[:, 2 * H:3 * H])
        o = jax.nn.sigmoid(g[:, 3 * H:])
        c = f * c + i * gg
        h = o * jnp.tanh(c)
    out_ref[:] = h


def _tc_lstm(x, noise2d, W_ih, W_hh, b_lstm):
    return pl.pallas_call(
        _lstm_body,
        grid=(NB,),
        in_specs=[
            pl.BlockSpec((R, F_IN), lambda i: (i, 0)),
            pl.BlockSpec((R, T * F_IN), lambda i: (i, 0)),
            pl.BlockSpec((4 * H, F_IN), lambda i: (0, 0)),
            pl.BlockSpec((4 * H, H), lambda i: (0, 0)),
            pl.BlockSpec((1, 4 * H), lambda i: (0, 0)),
        ],
        out_specs=pl.BlockSpec((R, H), lambda i: (i, 0)),
        out_shape=jax.ShapeDtypeStruct((N, H), jnp.float32),
        interpret=_INTERPRET,
    )(x, noise2d, W_ih, W_hh, b_lstm)


def _scale1_body(h_ref, w1_ref, dg0_ref, dg1_ref, tab_ref, dinv_ref):
    deg = dg0_ref[:] + dg1_ref[:] + 1.0
    dinv = lax.rsqrt(deg)
    y = jnp.dot(h_ref[:], w1_ref[:], preferred_element_type=jnp.float32)
    ys = y * dinv
    tab_ref[0] = ys[:, :H]
    tab_ref[1] = ys[:, H:]
    dinv_ref[:] = dinv


def _tc_scale1(h_last, W1, dg0, dg1):
    return pl.pallas_call(
        _scale1_body,
        grid=(NB,),
        in_specs=[
            pl.BlockSpec((R, H), lambda i: (i, 0)),
            pl.BlockSpec((H, G1), lambda i: (0, 0)),
            pl.BlockSpec((R, 1), lambda i: (i, 0)),
            pl.BlockSpec((R, 1), lambda i: (i, 0)),
        ],
        out_specs=[
            pl.BlockSpec((2, R, H), lambda i: (0, i, 0)),
            pl.BlockSpec((R, 1), lambda i: (i, 0)),
        ],
        out_shape=[
            jax.ShapeDtypeStruct((2, N, H), jnp.float32),
            jax.ShapeDtypeStruct((N, 1), jnp.float32),
        ],
        interpret=_INTERPRET,
    )(h_last, W1, dg0, dg1)


def _post1_body(a0_ref, a1_ref, ysa_ref, ysb_ref,
                dinv_ref, w2_ref, b1_ref, ys2_ref):
    dinv = dinv_ref[:]
    b1 = b1_ref[:]
    z1a = jax.nn.relu((a0_ref[:] + ysa_ref[:]) * dinv + b1[:, :H])
    z1b = jax.nn.relu((a1_ref[:] + ysb_ref[:]) * dinv + b1[:, H:])
    z1 = jnp.concatenate([z1a, z1b], axis=1)
    ys2 = jnp.dot(z1, w2_ref[:], preferred_element_type=jnp.float32) * dinv
    ys2_ref[:] = ys2


def _tc_post1(a0, a1, ysa, ysb, dinv, W2, b1):
    rblk = pl.BlockSpec((R, H), lambda i: (i, 0))
    return pl.pallas_call(
        _post1_body,
        grid=(NB,),
        in_specs=[rblk, rblk, rblk, rblk,
                  pl.BlockSpec((R, 1), lambda i: (i, 0)),
                  pl.BlockSpec((G1, G2), lambda i: (0, 0)),
                  pl.BlockSpec((1, G1), lambda i: (0, 0))],
        out_specs=pl.BlockSpec((R, G2), lambda i: (i, 0)),
        out_shape=jax.ShapeDtypeStruct((N, G2), jnp.float32),
        interpret=_INTERPRET,
    )(a0, a1, ysa, ysb, dinv, W2, b1)


def _final_body(a0_ref, a1_ref, ys2_ref, dinv_ref, b2_ref, wout_ref,
                bout_ref, out_ref, acc_ref):
    i = pl.program_id(0)
    z2 = jax.nn.relu((a0_ref[:] + a1_ref[:] + ys2_ref[:]) * dinv_ref[:]
                     + b2_ref[:])
    p = jnp.sum(z2, axis=0, keepdims=True)

    @pl.when(i == 0)
    def _():
        acc_ref[:] = p

    @pl.when(i != 0)
    def _():
        acc_ref[:] = acc_ref[:] + p

    @pl.when(i == NB - 1)
    def _():
        pooled = acc_ref[:] * (1.0 / N)
        out_ref[:] = jnp.dot(pooled, wout_ref[:],
                             preferred_element_type=jnp.float32) + bout_ref[:]


def _tc_final(a0, a1, ys2, dinv, b2, W_out, b_out2d):
    rblk = pl.BlockSpec((R, G2), lambda i: (i, 0))
    return pl.pallas_call(
        _final_body,
        grid=(NB,),
        in_specs=[rblk, rblk, rblk,
                  pl.BlockSpec((R, 1), lambda i: (i, 0)),
                  pl.BlockSpec((1, G2), lambda i: (0, 0)),
                  pl.BlockSpec((G2, 1), lambda i: (0, 0)),
                  pl.BlockSpec((1, 1), lambda i: (0, 0))],
        out_specs=pl.BlockSpec((1, 1), lambda i: (0, 0)),
        out_shape=jax.ShapeDtypeStruct((1, 1), jnp.float32),
        scratch_shapes=[pltpu.VMEM((1, G2), jnp.float32)],
        interpret=_INTERPRET,
    )(a0, a1, ys2, dinv, b2, W_out, b_out2d)


_SC_MESH = dict(core_axis_name="c", subcore_axis_name="s",
                num_cores=NC, num_subcores=NS)


def _sc_deg(dst2d, zeros128, ones128):
    mesh = plsc.VectorSubcoreMesh(**_SC_MESH)

    @functools.partial(
        pl.kernel,
        out_type=jax.ShapeDtypeStruct((NC, N_PAD, H), jnp.float32),
        mesh=mesh,
        scratch_types=[
            pltpu.VMEM((CB, EB), jnp.int32),
            pltpu.VMEM((EB, H), jnp.float32),
            pltpu.VMEM_SHARED((N_PAD, H), jnp.float32),
        ],
    )
    def k(dst_hbm, zeros_hbm, ones_hbm, out_hbm, idx_d, ones_v, acc):
        cid = lax.axis_index("c")
        sid = lax.axis_index("s")
        wid = cid * NS + sid
        pltpu.sync_copy(zeros_hbm.at[pl.ds(sid * ROWS_PER_TILE, ROWS_PER_TILE)],
                        acc.at[pl.ds(sid * ROWS_PER_TILE, ROWS_PER_TILE)])
        pltpu.sync_copy(ones_hbm, ones_v)
        plsc.subcore_barrier()
        bbase = wid * NBATCH

        def chunk(ch, carry):
            pltpu.sync_copy(dst_hbm.at[pl.ds(bbase + ch * CB, CB)], idx_d)

            def body(j, c2):
                pltpu.sync_copy(ones_v, acc.at[idx_d.at[j]], add=True)
                return c2

            lax.fori_loop(0, CB, body, carry)
            return carry

        lax.fori_loop(0, NBATCH // CB, chunk, 0)
        plsc.subcore_barrier()
        pltpu.sync_copy(
            acc.at[pl.ds(sid * ROWS_PER_TILE, ROWS_PER_TILE)],
            out_hbm.at[cid].at[pl.ds(sid * ROWS_PER_TILE, ROWS_PER_TILE)])

    return k(dst2d, zeros128, ones128)


CB = 40


def _agg_pipeline(table_view, src2d_hbm, dst2d_hbm, acc, idx_s, idx_d,
                  buf0, buf1, sem0, sem1, bbase, nbatch):
    def chunk(ch, carry):
        pltpu.sync_copy(src2d_hbm.at[pl.ds(bbase + ch * CB, CB)], idx_s)
        pltpu.sync_copy(dst2d_hbm.at[pl.ds(bbase + ch * CB, CB)], idx_d)
        pltpu.async_copy(table_view.at[idx_s.at[0]], buf0, sem0)

        def body(jj, c2):
            b0 = jj * 2
            pltpu.make_async_copy(table_view.at[idx_s.at[b0]],
                                  buf0, sem0).wait()
            pltpu.async_copy(table_view.at[idx_s.at[b0 + 1]], buf1, sem1)
            pltpu.sync_copy(buf0, acc.at[idx_d.at[b0]], add=True)
            pltpu.make_async_copy(table_view.at[idx_s.at[b0 + 1]],
                                  buf1, sem1).wait()

            @pl.when(b0 + 2 < CB)
            def _():
                pltpu.async_copy(table_view.at[idx_s.at[b0 + 2]], buf0, sem0)

            pltpu.sync_copy(buf1, acc.at[idx_d.at[b0 + 1]], add=True)
            return c2

        lax.fori_loop(0, CB // 2, body, carry)
        return carry

    lax.fori_loop(0, nbatch // CB, chunk, 0)


_AGG_SCRATCH = [
    pltpu.VMEM((CB, EB), jnp.int32),
    pltpu.VMEM((CB, EB), jnp.int32),
    pltpu.VMEM((EB, H), jnp.float32),
    pltpu.VMEM((EB, H), jnp.float32),
    pltpu.SemaphoreType.DMA,
    pltpu.SemaphoreType.DMA,
    pltpu.VMEM_SHARED((N_PAD, H), jnp.float32),
]


def _sc_agg_pair(tab2, src2d, dst2d, zeros128):
    nbatch = E_PAD // EB // NS
    mesh = plsc.VectorSubcoreMesh(**_SC_MESH)

    @functools.partial(
        pl.kernel,
        out_type=jax.ShapeDtypeStruct((NC, N_PAD, H), jnp.float32),
        mesh=mesh,
        scratch_types=_AGG_SCRATCH,
    )
    def k(tab_hbm, src_hbm, dst_hbm, zeros_hbm, out_hbm,
          idx_s, idx_d, buf0, buf1, sem0, sem1, acc):
        cid = lax.axis_index("c")
        sid = lax.axis_index("s")
        pltpu.sync_copy(zeros_hbm.at[pl.ds(sid * ROWS_PER_TILE, ROWS_PER_TILE)],
                        acc.at[pl.ds(sid * ROWS_PER_TILE, ROWS_PER_TILE)])
        plsc.subcore_barrier()
        _agg_pipeline(tab_hbm.at[cid], src_hbm, dst_hbm, acc, idx_s, idx_d,
                      buf0, buf1, sem0, sem1, sid * nbatch, nbatch)
        plsc.subcore_barrier()
        pltpu.sync_copy(
            acc.at[pl.ds(sid * ROWS_PER_TILE, ROWS_PER_TILE)],
            out_hbm.at[cid].at[pl.ds(sid * ROWS_PER_TILE, ROWS_PER_TILE)])

    return k(tab2, src2d, dst2d, zeros128)


def _sc_agg(table, src2d, dst2d, zeros128):
    nbatch = NBATCH
    mesh = plsc.VectorSubcoreMesh(**_SC_MESH)

    @functools.partial(
        pl.kernel,
        out_type=jax.ShapeDtypeStruct((NC, N_PAD, H), jnp.float32),
        mesh=mesh,
        scratch_types=_AGG_SCRATCH,
    )
    def k(tab_hbm, src_hbm, dst_hbm, zeros_hbm, out_hbm,
          idx_s, idx_d, buf0, buf1, sem0, sem1, acc):
        cid = lax.axis_index("c")
        sid = lax.axis_index("s")
        wid = cid * NS + sid
        pltpu.sync_copy(zeros_hbm.at[pl.ds(sid * ROWS_PER_TILE, ROWS_PER_TILE)],
                        acc.at[pl.ds(sid * ROWS_PER_TILE, ROWS_PER_TILE)])
        plsc.subcore_barrier()
        _agg_pipeline(tab_hbm, src_hbm, dst_hbm, acc, idx_s, idx_d,
                      buf0, buf1, sem0, sem1, wid * nbatch, nbatch)
        plsc.subcore_barrier()
        pltpu.sync_copy(
            acc.at[pl.ds(sid * ROWS_PER_TILE, ROWS_PER_TILE)],
            out_hbm.at[cid].at[pl.ds(sid * ROWS_PER_TILE, ROWS_PER_TILE)])

    return k(table, src2d, dst2d, zeros128)


def kernel(x, W_ih, W_hh, b_ih, b_hh, W1, b1, W2, b2, W_out, b_out,
           edge_index):
    noise2d = (jax.random.normal(jax.random.key(42), (N, T, F_IN),
                                 dtype=jnp.float32) * 0.1
               ).reshape(N, T * F_IN).astype(jnp.bfloat16)
    x_bf = x.astype(jnp.bfloat16)
    wih_bf = W_ih.astype(jnp.bfloat16)
    whh_bf = W_hh.astype(jnp.bfloat16)
    b_lstm = (b_ih + b_hh).reshape(1, 4 * H)

    src = edge_index[0]
    dst = edge_index[1]
    src_p = jnp.concatenate(
        [src, jnp.zeros((E_PAD - E,), jnp.int32)])
    dst_p = jnp.concatenate(
        [dst, jnp.full((E_PAD - E,), N_PAD - 1, jnp.int32)])

    src2d = src_p.reshape(E_PAD // EB, EB)
    dst2d = dst_p.reshape(E_PAD // EB, EB)

    zeros128 = jnp.zeros((N_PAD, H), jnp.float32)
    ones128 = jnp.ones((EB, H), jnp.float32)

    degp = _sc_deg(dst2d, zeros128, ones128)
    dg0 = degp[0, :N, 0:1]
    dg1 = degp[1, :N, 0:1]

    h_last = _tc_lstm(x_bf, noise2d, wih_bf, whh_bf, b_lstm)

    tab2, dinv = _tc_scale1(h_last, W1, dg0, dg1)

    agg1 = _sc_agg_pair(tab2, src2d, dst2d, zeros128)

    ys2 = _tc_post1(agg1[0, :N], agg1[1, :N], tab2[0, :N], tab2[1, :N],
                    dinv, W2, b1.reshape(1, G1))

    agg2 = _sc_agg(ys2, src2d, dst2d, zeros128)

    out = _tc_final(agg2[0, :N], agg2[1, :N], ys2, dinv,
                    b2.reshape(1, G2), W_out, b_out.reshape(1, 1))
    return out.reshape(1)


# speedup vs baseline: 1.2439x; 1.0024x over previous
import functools

import jax
import jax.numpy as jnp
from jax import lax
from jax.experimental import pallas as pl
from jax.experimental.pallas import tpu as pltpu
from jax.experimental.pallas import tpu_sc as plsc

N = 10000
E = 320000
T = 16
F_IN = 128
H = 128
G1 = 256
G2 = 128

NC = 2
NS = 16
NW = NC * NS
N_PAD = 10240
E_PAD = 327680
E_PER_TILE = E_PAD // NW
EB = 64
NBATCH = E_PER_TILE // EB
ROWS_PER_TILE = N_PAD // NS

R = 1000
NB = N // R

_INTERPRET = False


def _dot_t(a, w):
    return lax.dot_general(a, w, (((1,), (1,)), ((), ())),
                           preferred_element_type=jnp.float32)


def _lstm_body(x_ref, n_ref, wih_ref, whh_ref, b_ref, out_ref):
    x = x_ref[:]
    wih = wih_ref[:]
    whh = whh_ref[:]
    xw = _dot_t(x, wih) + b_ref[:]
    h = jnp.zeros((R, H), jnp.float32)
    c = jnp.zeros((R, H), jnp.float32)
    for t in range(T):
        nt = n_ref[:, t * F_IN:(t + 1) * F_IN]
        g = xw + _dot_t(nt, wih) + _dot_t(h.astype(jnp.bfloat16), whh)
        i = jax.nn.sigmoid(g[:, :H])
        f = jax.nn.sigmoid(g[:, H:2 * H])
        gg = jnp.tanh(g[:, 2 * H:3 * H])
        o = jax.nn.sigmoid(g[:, 3 * H:])
        c = f * c + i * gg
        h = o * jnp.tanh(c)
    out_ref[:] = h


def _tc_lstm(x, noise2d, W_ih, W_hh, b_lstm):
    return pl.pallas_call(
        _lstm_body,
        grid=(NB,),
        in_specs=[
            pl.BlockSpec((R, F_IN), lambda i: (i, 0)),
            pl.BlockSpec((R, T * F_IN), lambda i: (i, 0)),
            pl.BlockSpec((4 * H, F_IN), lambda i: (0, 0)),
            pl.BlockSpec((4 * H, H), lambda i: (0, 0)),
            pl.BlockSpec((1, 4 * H), lambda i: (0, 0)),
        ],
        out_specs=pl.BlockSpec((R, H), lambda i: (i, 0)),
        out_shape=jax.ShapeDtypeStruct((N, H), jnp.float32),
        interpret=_INTERPRET,
    )(x, noise2d, W_ih, W_hh, b_lstm)


def _scale1_body(h_ref, w1_ref, dg0_ref, dg1_ref, tab_ref, dinv_ref):
    deg = dg0_ref[:] + dg1_ref[:] + 1.0
    dinv = lax.rsqrt(deg)
    y = jnp.dot(h_ref[:], w1_ref[:], preferred_element_type=jnp.float32)
    ys = y * dinv
    tab_ref[0] = ys[:, :H]
    tab_ref[1] = ys[:, H:]
    dinv_ref[:] = dinv


def _tc_scale1(h_last, W1, dg0, dg1):
    return pl.pallas_call(
        _scale1_body,
        grid=(NB,),
        in_specs=[
            pl.BlockSpec((R, H), lambda i: (i, 0)),
            pl.BlockSpec((H, G1), lambda i: (0, 0)),
            pl.BlockSpec((R, 1), lambda i: (i, 0)),
            pl.BlockSpec((R, 1), lambda i: (i, 0)),
        ],
        out_specs=[
            pl.BlockSpec((2, R, H), lambda i: (0, i, 0)),
            pl.BlockSpec((R, 1), lambda i: (i, 0)),
        ],
        out_shape=[
            jax.ShapeDtypeStruct((2, N, H), jnp.float32),
            jax.ShapeDtypeStruct((N, 1), jnp.float32),
        ],
        interpret=_INTERPRET,
    )(h_last, W1, dg0, dg1)


def _post1_body(a0_ref, a1_ref, ysa_ref, ysb_ref,
                dinv_ref, w2_ref, b1_ref, ys2_ref):
    dinv = dinv_ref[:]
    b1 = b1_ref[:]
    z1a = jax.nn.relu((a0_ref[:] + ysa_ref[:]) * dinv + b1[:, :H])
    z1b = jax.nn.relu((a1_ref[:] + ysb_ref[:]) * dinv + b1[:, H:])
    z1 = jnp.concatenate([z1a, z1b], axis=1)
    ys2 = jnp.dot(z1, w2_ref[:], preferred_element_type=jnp.float32) * dinv
    ys2_ref[:] = ys2


def _tc_post1(a0, a1, ysa, ysb, dinv, W2, b1):
    rblk = pl.BlockSpec((R, H), lambda i: (i, 0))
    return pl.pallas_call(
        _post1_body,
        grid=(NB,),
        in_specs=[rblk, rblk, rblk, rblk,
                  pl.BlockSpec((R, 1), lambda i: (i, 0)),
                  pl.BlockSpec((G1, G2), lambda i: (0, 0)),
                  pl.BlockSpec((1, G1), lambda i: (0, 0))],
        out_specs=pl.BlockSpec((R, G2), lambda i: (i, 0)),
        out_shape=jax.ShapeDtypeStruct((N, G2), jnp.float32),
        interpret=_INTERPRET,
    )(a0, a1, ysa, ysb, dinv, W2, b1)


def _final_body(a0_ref, a1_ref, ys2_ref, dinv_ref, b2_ref, wout_ref,
                bout_ref, out_ref, acc_ref):
    i = pl.program_id(0)
    z2 = jax.nn.relu((a0_ref[:] + a1_ref[:] + ys2_ref[:]) * dinv_ref[:]
                     + b2_ref[:])
    p = jnp.sum(z2, axis=0, keepdims=True)

    @pl.when(i == 0)
    def _():
        acc_ref[:] = p

    @pl.when(i != 0)
    def _():
        acc_ref[:] = acc_ref[:] + p

    @pl.when(i == NB - 1)
    def _():
        pooled = acc_ref[:] * (1.0 / N)
        out_ref[:] = jnp.dot(pooled, wout_ref[:],
                             preferred_element_type=jnp.float32) + bout_ref[:]


def _tc_final(a0, a1, ys2, dinv, b2, W_out, b_out2d):
    rblk = pl.BlockSpec((R, G2), lambda i: (i, 0))
    return pl.pallas_call(
        _final_body,
        grid=(NB,),
        in_specs=[rblk, rblk, rblk,
                  pl.BlockSpec((R, 1), lambda i: (i, 0)),
                  pl.BlockSpec((1, G2), lambda i: (0, 0)),
                  pl.BlockSpec((G2, 1), lambda i: (0, 0)),
                  pl.BlockSpec((1, 1), lambda i: (0, 0))],
        out_specs=pl.BlockSpec((1, 1), lambda i: (0, 0)),
        out_shape=jax.ShapeDtypeStruct((1, 1), jnp.float32),
        scratch_shapes=[pltpu.VMEM((1, G2), jnp.float32)],
        interpret=_INTERPRET,
    )(a0, a1, ys2, dinv, b2, W_out, b_out2d)


_SC_MESH = dict(core_axis_name="c", subcore_axis_name="s",
                num_cores=NC, num_subcores=NS)


def _sc_deg(dst2d, zeros128, ones128):
    mesh = plsc.VectorSubcoreMesh(**_SC_MESH)

    @functools.partial(
        pl.kernel,
        out_type=jax.ShapeDtypeStruct((NC, N_PAD, H), jnp.float32),
        mesh=mesh,
        scratch_types=[
            pltpu.VMEM((CB, EB), jnp.int32),
            pltpu.VMEM((EB, H), jnp.float32),
            pltpu.VMEM_SHARED((N_PAD, H), jnp.float32),
        ],
    )
    def k(dst_hbm, zeros_hbm, ones_hbm, out_hbm, idx_d, ones_v, acc):
        cid = lax.axis_index("c")
        sid = lax.axis_index("s")
        wid = cid * NS + sid
        pltpu.sync_copy(zeros_hbm.at[pl.ds(sid * ROWS_PER_TILE, ROWS_PER_TILE)],
                        acc.at[pl.ds(sid * ROWS_PER_TILE, ROWS_PER_TILE)])
        pltpu.sync_copy(ones_hbm, ones_v)
        plsc.subcore_barrier()
        bbase = wid * NBATCH

        def chunk(ch, carry):
            pltpu.sync_copy(dst_hbm.at[pl.ds(bbase + ch * CB, CB)], idx_d)

            def body(j, c2):
                pltpu.sync_copy(ones_v, acc.at[idx_d.at[j]], add=True)
                return c2

            lax.fori_loop(0, CB, body, carry)
            return carry

        lax.fori_loop(0, NBATCH // CB, chunk, 0)
        plsc.subcore_barrier()
        pltpu.sync_copy(
            acc.at[pl.ds(sid * ROWS_PER_TILE, ROWS_PER_TILE)],
            out_hbm.at[cid].at[pl.ds(sid * ROWS_PER_TILE, ROWS_PER_TILE)])

    return k(dst2d, zeros128, ones128)


CB = 40


NBUF = 4


def _agg_pipeline(table_view, src2d_hbm, dst2d_hbm, acc, idx_s, idx_d,
                  bufs, sems, bbase, nbatch):
    def chunk(ch, carry):
        pltpu.sync_copy(src2d_hbm.at[pl.ds(bbase + ch * CB, CB)], idx_s)
        pltpu.sync_copy(dst2d_hbm.at[pl.ds(bbase + ch * CB, CB)], idx_d)
        for b in range(NBUF - 1):
            pltpu.async_copy(table_view.at[idx_s.at[b]], bufs[b], sems[b])

        def body(jj, c2):
            for k in range(NBUF):
                b = jj * NBUF + k
                pltpu.make_async_copy(table_view.at[idx_s.at[b]],
                                      bufs[k], sems[k]).wait()

                @pl.when(b + NBUF - 1 < CB)
                def _():
                    kn = (k + NBUF - 1) % NBUF
                    pltpu.async_copy(table_view.at[idx_s.at[b + NBUF - 1]],
                                     bufs[kn], sems[kn])

                pltpu.sync_copy(bufs[k], acc.at[idx_d.at[b]], add=True)
            return c2

        lax.fori_loop(0, CB // NBUF, body, carry)
        return carry

    lax.fori_loop(0, nbatch // CB, chunk, 0)


_AGG_SCRATCH = [
    pltpu.VMEM((CB, EB), jnp.int32),
    pltpu.VMEM((CB, EB), jnp.int32),
    [pltpu.VMEM((EB, H), jnp.float32) for _ in range(NBUF)],
    [pltpu.SemaphoreType.DMA for _ in range(NBUF)],
    pltpu.VMEM_SHARED((N_PAD, H), jnp.float32),
]


def _sc_agg_pair(tab2, src2d, dst2d, zeros128):
    nbatch = E_PAD // EB // NS
    mesh = plsc.VectorSubcoreMesh(**_SC_MESH)

    @functools.partial(
        pl.kernel,
        out_type=jax.ShapeDtypeStruct((NC, N_PAD, H), jnp.float32),
        mesh=mesh,
        scratch_types=_AGG_SCRATCH,
    )
    def k(tab_hbm, src_hbm, dst_hbm, zeros_hbm, out_hbm,
          idx_s, idx_d, bufs, sems, acc):
        cid = lax.axis_index("c")
        sid = lax.axis_index("s")
        pltpu.sync_copy(zeros_hbm.at[pl.ds(sid * ROWS_PER_TILE, ROWS_PER_TILE)],
                        acc.at[pl.ds(sid * ROWS_PER_TILE, ROWS_PER_TILE)])
        plsc.subcore_barrier()
        _agg_pipeline(tab_hbm.at[cid], src_hbm, dst_hbm, acc, idx_s, idx_d,
                      bufs, sems, sid * nbatch, nbatch)
        plsc.subcore_barrier()
        pltpu.sync_copy(
            acc.at[pl.ds(sid * ROWS_PER_TILE, ROWS_PER_TILE)],
            out_hbm.at[cid].at[pl.ds(sid * ROWS_PER_TILE, ROWS_PER_TILE)])

    return k(tab2, src2d, dst2d, zeros128)


def _sc_agg(table, src2d, dst2d, zeros128):
    nbatch = NBATCH
    mesh = plsc.VectorSubcoreMesh(**_SC_MESH)

    @functools.partial(
        pl.kernel,
        out_type=jax.ShapeDtypeStruct((NC, N_PAD, H), jnp.float32),
        mesh=mesh,
        scratch_types=_AGG_SCRATCH,
    )
    def k(tab_hbm, src_hbm, dst_hbm, zeros_hbm, out_hbm,
          idx_s, idx_d, bufs, sems, acc):
        cid = lax.axis_index("c")
        sid = lax.axis_index("s")
        wid = cid * NS + sid
        pltpu.sync_copy(zeros_hbm.at[pl.ds(sid * ROWS_PER_TILE, ROWS_PER_TILE)],
                        acc.at[pl.ds(sid * ROWS_PER_TILE, ROWS_PER_TILE)])
        plsc.subcore_barrier()
        _agg_pipeline(tab_hbm, src_hbm, dst_hbm, acc, idx_s, idx_d,
                      bufs, sems, wid * nbatch, nbatch)
        plsc.subcore_barrier()
        pltpu.sync_copy(
            acc.at[pl.ds(sid * ROWS_PER_TILE, ROWS_PER_TILE)],
            out_hbm.at[cid].at[pl.ds(sid * ROWS_PER_TILE, ROWS_PER_TILE)])

    return k(table, src2d, dst2d, zeros128)


def kernel(x, W_ih, W_hh, b_ih, b_hh, W1, b1, W2, b2, W_out, b_out,
           edge_index):
    noise2d = (jax.random.normal(jax.random.key(42), (N, T, F_IN),
                                 dtype=jnp.float32) * 0.1
               ).reshape(N, T * F_IN).astype(jnp.bfloat16)
    x_bf = x.astype(jnp.bfloat16)
    wih_bf = W_ih.astype(jnp.bfloat16)
    whh_bf = W_hh.astype(jnp.bfloat16)
    b_lstm = (b_ih + b_hh).reshape(1, 4 * H)

    src = edge_index[0]
    dst = edge_index[1]
    src_p = jnp.concatenate(
        [src, jnp.zeros((E_PAD - E,), jnp.int32)])
    dst_p = jnp.concatenate(
        [dst, jnp.full((E_PAD - E,), N_PAD - 1, jnp.int32)])

    src2d = src_p.reshape(E_PAD // EB, EB)
    dst2d = dst_p.reshape(E_PAD // EB, EB)

    zeros128 = jnp.zeros((N_PAD, H), jnp.float32)
    ones128 = jnp.ones((EB, H), jnp.float32)

    degp = _sc_deg(dst2d, zeros128, ones128)
    dg0 = degp[0, :N, 0:1]
    dg1 = degp[1, :N, 0:1]

    h_last = _tc_lstm(x_bf, noise2d, wih_bf, whh_bf, b_lstm)

    tab2, dinv = _tc_scale1(h_last, W1, dg0, dg1)

    agg1 = _sc_agg_pair(tab2, src2d, dst2d, zeros128)

    ys2 = _tc_post1(agg1[0, :N], agg1[1, :N], tab2[0, :N], tab2[1, :N],
                    dinv, W2, b1.reshape(1, G1))

    agg2 = _sc_agg(ys2, src2d, dst2d, zeros128)

    out = _tc_final(agg2[0, :N], agg2[1, :N], ys2, dinv,
                    b2.reshape(1, G2), W_out, b_out.reshape(1, 1))
    return out.reshape(1)


# speedup vs baseline: 1.2892x; 1.0364x over previous
import functools

import jax
import jax.numpy as jnp
from jax import lax
from jax.experimental import pallas as pl
from jax.experimental.pallas import tpu as pltpu
from jax.experimental.pallas import tpu_sc as plsc

N = 10000
E = 320000
T = 16
F_IN = 128
H = 128
G1 = 256
G2 = 128

NC = 2
NS = 16
NW = NC * NS
N_PAD = 10240
E_PAD = 327680
E_PER_TILE = E_PAD // NW
EB = 64
NBATCH = E_PER_TILE // EB
ROWS_PER_TILE = N_PAD // NS

R = 1000
NB = N // R

_INTERPRET = False


def _dot_t(a, w):
    return lax.dot_general(a, w, (((1,), (1,)), ((), ())),
                           preferred_element_type=jnp.float32)


def _lstm_body(x_ref, n_ref, wih_ref, whh_ref, b_ref, w1_ref,
               dg0_ref, dg1_ref, tab_ref, dinv_ref):
    x = x_ref[:]
    wih = wih_ref[:]
    whh = whh_ref[:]
    xw = _dot_t(x, wih) + b_ref[:]
    h = jnp.zeros((R, H), jnp.float32)
    c = jnp.zeros((R, H), jnp.float32)
    for t in range(T):
        nt = n_ref[:, t * F_IN:(t + 1) * F_IN]
        g = xw + _dot_t(nt, wih) + _dot_t(h.astype(jnp.bfloat16), whh)
        i = jax.nn.sigmoid(g[:, :H])
        f = jax.nn.sigmoid(g[:, H:2 * H])
        gg = jnp.tanh(g[:, 2 * H:3 * H])
        o = jax.nn.sigmoid(g[:, 3 * H:])
        c = f * c + i * gg
        h = o * jnp.tanh(c)
    deg = dg0_ref[:] + dg1_ref[:] + 1.0
    dinv = lax.rsqrt(deg)
    y = jnp.dot(h, w1_ref[:], preferred_element_type=jnp.float32)
    ys = y * dinv
    tab_ref[0] = ys[:, :H]
    tab_ref[1] = ys[:, H:]
    dinv_ref[:] = dinv


def _tc_lstm_scale1(x, noise2d, W_ih, W_hh, b_lstm, W1, dg0, dg1):
    return pl.pallas_call(
        _lstm_body,
        grid=(NB,),
        in_specs=[
            pl.BlockSpec((R, F_IN), lambda i: (i, 0)),
            pl.BlockSpec((R, T * F_IN), lambda i: (i, 0)),
            pl.BlockSpec((4 * H, F_IN), lambda i: (0, 0)),
            pl.BlockSpec((4 * H, H), lambda i: (0, 0)),
            pl.BlockSpec((1, 4 * H), lambda i: (0, 0)),
            pl.BlockSpec((H, G1), lambda i: (0, 0)),
            pl.BlockSpec((R, 1), lambda i: (i, 0)),
            pl.BlockSpec((R, 1), lambda i: (i, 0)),
        ],
        out_specs=[
            pl.BlockSpec((2, R, H), lambda i: (0, i, 0)),
            pl.BlockSpec((R, 1), lambda i: (i, 0)),
        ],
        out_shape=[
            jax.ShapeDtypeStruct((2, N, H), jnp.float32),
            jax.ShapeDtypeStruct((N, 1), jnp.float32),
        ],
        interpret=_INTERPRET,
    )(x, noise2d, W_ih, W_hh, b_lstm, W1, dg0, dg1)


def _post1_body(agg_ref0, agg_ref1, tab_ref0, tab_ref1,
                dinv_ref, w2_ref, b1_ref, ys2_ref):
    dinv = dinv_ref[:]
    b1 = b1_ref[:]
    z1a = jax.nn.relu((agg_ref0[0] + tab_ref0[0]) * dinv + b1[:, :H])
    z1b = jax.nn.relu((agg_ref1[0] + tab_ref1[0]) * dinv + b1[:, H:])
    z1 = jnp.concatenate([z1a, z1b], axis=1)
    ys2 = jnp.dot(z1, w2_ref[:], preferred_element_type=jnp.float32) * dinv
    ys2_ref[:] = ys2


def _tc_post1(agg1, tab2, dinv, W2, b1):
    blk0 = pl.BlockSpec((1, R, H), lambda i: (0, i, 0))
    blk1 = pl.BlockSpec((1, R, H), lambda i: (1, i, 0))
    return pl.pallas_call(
        _post1_body,
        grid=(NB,),
        in_specs=[blk0, blk1, blk0, blk1,
                  pl.BlockSpec((R, 1), lambda i: (i, 0)),
                  pl.BlockSpec((G1, G2), lambda i: (0, 0)),
                  pl.BlockSpec((1, G1), lambda i: (0, 0))],
        out_specs=pl.BlockSpec((R, G2), lambda i: (i, 0)),
        out_shape=jax.ShapeDtypeStruct((N, G2), jnp.float32),
        interpret=_INTERPRET,
    )(agg1, agg1, tab2, tab2, dinv, W2, b1)


def _final_body(a0_ref, a1_ref, ys2_ref, dinv_ref, b2_ref, wout_ref,
                bout_ref, out_ref, acc_ref):
    i = pl.program_id(0)
    z2 = jax.nn.relu((a0_ref[0] + a1_ref[0] + ys2_ref[:]) * dinv_ref[:]
                     + b2_ref[:])
    p = jnp.sum(z2, axis=0, keepdims=True)

    @pl.when(i == 0)
    def _():
        acc_ref[:] = p

    @pl.when(i != 0)
    def _():
        acc_ref[:] = acc_ref[:] + p

    @pl.when(i == NB - 1)
    def _():
        pooled = acc_ref[:] * (1.0 / N)
        out_ref[:] = jnp.dot(pooled, wout_ref[:],
                             preferred_element_type=jnp.float32) + bout_ref[:]


def _tc_final(agg2, ys2, dinv, b2, W_out, b_out2d):
    blk0 = pl.BlockSpec((1, R, G2), lambda i: (0, i, 0))
    blk1 = pl.BlockSpec((1, R, G2), lambda i: (1, i, 0))
    return pl.pallas_call(
        _final_body,
        grid=(NB,),
        in_specs=[blk0, blk1,
                  pl.BlockSpec((R, G2), lambda i: (i, 0)),
                  pl.BlockSpec((R, 1), lambda i: (i, 0)),
                  pl.BlockSpec((1, G2), lambda i: (0, 0)),
                  pl.BlockSpec((G2, 1), lambda i: (0, 0)),
                  pl.BlockSpec((1, 1), lambda i: (0, 0))],
        out_specs=pl.BlockSpec((1, 1), lambda i: (0, 0)),
        out_shape=jax.ShapeDtypeStruct((1, 1), jnp.float32),
        scratch_shapes=[pltpu.VMEM((1, G2), jnp.float32)],
        interpret=_INTERPRET,
    )(agg2, agg2, ys2, dinv, b2, W_out, b_out2d)


_SC_MESH = dict(core_axis_name="c", subcore_axis_name="s",
                num_cores=NC, num_subcores=NS)


def _sc_deg(dst2d, zeros128, ones128):
    mesh = plsc.VectorSubcoreMesh(**_SC_MESH)

    @functools.partial(
        pl.kernel,
        out_type=jax.ShapeDtypeStruct((NC, N_PAD, H), jnp.float32),
        mesh=mesh,
        scratch_types=[
            pltpu.VMEM((CB, EB), jnp.int32),
            pltpu.VMEM((EB, H), jnp.float32),
            pltpu.VMEM_SHARED((N_PAD, H), jnp.float32),
        ],
    )
    def k(dst_hbm, zeros_hbm, ones_hbm, out_hbm, idx_d, ones_v, acc):
        cid = lax.axis_index("c")
        sid = lax.axis_index("s")
        wid = cid * NS + sid
        pltpu.sync_copy(zeros_hbm.at[pl.ds(sid * ROWS_PER_TILE, ROWS_PER_TILE)],
                        acc.at[pl.ds(sid * ROWS_PER_TILE, ROWS_PER_TILE)])
        pltpu.sync_copy(ones_hbm, ones_v)
        plsc.subcore_barrier()
        bbase = wid * NBATCH

        def chunk(ch, carry):
            pltpu.sync_copy(dst_hbm.at[pl.ds(bbase + ch * CB, CB)], idx_d)

            def body(j, c2):
                pltpu.sync_copy(ones_v, acc.at[idx_d.at[j]], add=True)
                return c2

            lax.fori_loop(0, CB, body, carry)
            return carry

        lax.fori_loop(0, NBATCH // CB, chunk, 0)
        plsc.subcore_barrier()
        pltpu.sync_copy(
            acc.at[pl.ds(sid * ROWS_PER_TILE, ROWS_PER_TILE)],
            out_hbm.at[cid].at[pl.ds(sid * ROWS_PER_TILE, ROWS_PER_TILE)])

    return k(dst2d, zeros128, ones128)


CB = 40


NBUF = 4


def _agg_pipeline(table_view, src2d_hbm, dst2d_hbm, acc, idx_s, idx_d,
                  bufs, sems, bbase, nbatch):
    def chunk(ch, carry):
        pltpu.sync_copy(src2d_hbm.at[pl.ds(bbase + ch * CB, CB)], idx_s)
        pltpu.sync_copy(dst2d_hbm.at[pl.ds(bbase + ch * CB, CB)], idx_d)
        for b in range(NBUF - 1):
            pltpu.async_copy(table_view.at[idx_s.at[b]], bufs[b], sems[b])

        def body(jj, c2):
            for k in range(NBUF):
                b = jj * NBUF + k
                pltpu.make_async_copy(table_view.at[idx_s.at[b]],
                                      bufs[k], sems[k]).wait()

                @pl.when(b + NBUF - 1 < CB)
                def _():
                    kn = (k + NBUF - 1) % NBUF
                    pltpu.async_copy(table_view.at[idx_s.at[b + NBUF - 1]],
                                     bufs[kn], sems[kn])

                pltpu.sync_copy(bufs[k], acc.at[idx_d.at[b]], add=True)
            return c2

        lax.fori_loop(0, CB // NBUF, body, carry)
        return carry

    lax.fori_loop(0, nbatch // CB, chunk, 0)


_AGG_SCRATCH = [
    pltpu.VMEM((CB, EB), jnp.int32),
    pltpu.VMEM((CB, EB), jnp.int32),
    [pltpu.VMEM((EB, H), jnp.float32) for _ in range(NBUF)],
    [pltpu.SemaphoreType.DMA for _ in range(NBUF)],
    pltpu.VMEM_SHARED((N_PAD, H), jnp.float32),
]


def _sc_agg_pair(tab2, src2d, dst2d, zeros128):
    nbatch = E_PAD // EB // NS
    mesh = plsc.VectorSubcoreMesh(**_SC_MESH)

    @functools.partial(
        pl.kernel,
        out_type=jax.ShapeDtypeStruct((NC, N_PAD, H), jnp.float32),
        mesh=mesh,
        scratch_types=_AGG_SCRATCH,
    )
    def k(tab_hbm, src_hbm, dst_hbm, zeros_hbm, out_hbm,
          idx_s, idx_d, bufs, sems, acc):
        cid = lax.axis_index("c")
        sid = lax.axis_index("s")
        pltpu.sync_copy(zeros_hbm.at[pl.ds(sid * ROWS_PER_TILE, ROWS_PER_TILE)],
                        acc.at[pl.ds(sid * ROWS_PER_TILE, ROWS_PER_TILE)])
        plsc.subcore_barrier()
        _agg_pipeline(tab_hbm.at[cid], src_hbm, dst_hbm, acc, idx_s, idx_d,
                      bufs, sems, sid * nbatch, nbatch)
        plsc.subcore_barrier()
        pltpu.sync_copy(
            acc.at[pl.ds(sid * ROWS_PER_TILE, ROWS_PER_TILE)],
            out_hbm.at[cid].at[pl.ds(sid * ROWS_PER_TILE, ROWS_PER_TILE)])

    return k(tab2, src2d, dst2d, zeros128)


def _sc_agg(table, src2d, dst2d, zeros128):
    nbatch = NBATCH
    mesh = plsc.VectorSubcoreMesh(**_SC_MESH)

    @functools.partial(
        pl.kernel,
        out_type=jax.ShapeDtypeStruct((NC, N_PAD, H), jnp.float32),
        mesh=mesh,
        scratch_types=_AGG_SCRATCH,
    )
    def k(tab_hbm, src_hbm, dst_hbm, zeros_hbm, out_hbm,
          idx_s, idx_d, bufs, sems, acc):
        cid = lax.axis_index("c")
        sid = lax.axis_index("s")
        wid = cid * NS + sid
        pltpu.sync_copy(zeros_hbm.at[pl.ds(sid * ROWS_PER_TILE, ROWS_PER_TILE)],
                        acc.at[pl.ds(sid * ROWS_PER_TILE, ROWS_PER_TILE)])
        plsc.subcore_barrier()
        _agg_pipeline(tab_hbm, src_hbm, dst_hbm, acc, idx_s, idx_d,
                      bufs, sems, wid * nbatch, nbatch)
        plsc.subcore_barrier()
        pltpu.sync_copy(
            acc.at[pl.ds(sid * ROWS_PER_TILE, ROWS_PER_TILE)],
            out_hbm.at[cid].at[pl.ds(sid * ROWS_PER_TILE, ROWS_PER_TILE)])

    return k(table, src2d, dst2d, zeros128)


def kernel(x, W_ih, W_hh, b_ih, b_hh, W1, b1, W2, b2, W_out, b_out,
           edge_index):
    noise2d = (jax.random.normal(jax.random.key(42), (N, T, F_IN),
                                 dtype=jnp.float32) * 0.1
               ).reshape(N, T * F_IN).astype(jnp.bfloat16)
    x_bf = x.astype(jnp.bfloat16)
    wih_bf = W_ih.astype(jnp.bfloat16)
    whh_bf = W_hh.astype(jnp.bfloat16)
    b_lstm = (b_ih + b_hh).reshape(1, 4 * H)

    src = edge_index[0]
    dst = edge_index[1]
    src_p = jnp.concatenate(
        [src, jnp.zeros((E_PAD - E,), jnp.int32)])
    dst_p = jnp.concatenate(
        [dst, jnp.full((E_PAD - E,), N_PAD - 1, jnp.int32)])

    src2d = src_p.reshape(E_PAD // EB, EB)
    dst2d = dst_p.reshape(E_PAD // EB, EB)

    zeros128 = jnp.zeros((N_PAD, H), jnp.float32)
    ones128 = jnp.ones((EB, H), jnp.float32)

    degp = _sc_deg(dst2d, zeros128, ones128)
    dg0 = degp[0, :N, 0:1]
    dg1 = degp[1, :N, 0:1]

    tab2, dinv = _tc_lstm_scale1(x_bf, noise2d, wih_bf, whh_bf, b_lstm,
                                 W1, dg0, dg1)

    agg1 = _sc_agg_pair(tab2, src2d, dst2d, zeros128)

    ys2 = _tc_post1(agg1, tab2, dinv, W2, b1.reshape(1, G1))

    agg2 = _sc_agg(ys2, src2d, dst2d, zeros128)

    out = _tc_final(agg2, ys2, dinv,
                    b2.reshape(1, G2), W_out, b_out.reshape(1, 1))
    return out.reshape(1)
